# baseline (speedup 1.0000x reference)
import functools

import jax
import jax.numpy as jnp
from jax import lax
from jax.experimental import pallas as pl
from jax.experimental.pallas import tpu as pltpu
from jax.experimental.pallas import tpu_sc as plsc

N = 10000
E = 320000
K = 128
NC, NS = 2, 16
NW = NC * NS
CPW = 79
EP = NW * CPW * K
CPT = EP // (NS * K)
RF = 632
RL = N - (NS - 1) * RF
NBUF = 4


def _sc_mesh():
    return plsc.VectorSubcoreMesh(core_axis_name="c", subcore_axis_name="s")


def _pipeline(n_chunks, gather, wait, scatter):
    for b in range(NBUF - 1):
        gather(b, b)

    def body(i, carry):
        j = i * NBUF
        for t in range(NBUF):
            f = j + t + NBUF - 1
            if t == 0:
                gather(f, (NBUF - 1) % NBUF)
            else:
                @pl.when(f < n_chunks)
                def _(f=f, t=t):
                    gather(f, t - 1)
            wait(t)
            scatter(j + t, t)
        return carry

    lax.fori_loop(0, n_chunks // NBUF, body, 0)
    r = n_chunks % NBUF
    for t in range(r):
        j = n_chunks - r + t
        wait(t)
        scatter(j, t)


def _untiled():
    return pltpu.CompilerParams(use_tc_tiling_on_sc=False,
                                needs_layout_passes=False)


def _deg_call(dst3):
    @functools.partial(
        pl.kernel,
        out_type=jax.ShapeDtypeStruct((NW * N,), jnp.float32),
        mesh=_sc_mesh(),
        compiler_params=_untiled(),
        scratch_types=[
            pltpu.VMEM((CPW, K), jnp.int32),
            pltpu.VMEM((N + 16,), jnp.float32),
        ],
    )
    def deg(dst_hbm, out_hbm, idx_v, deg_v):
        c = lax.axis_index("c")
        s = lax.axis_index("s")
        wid = s * NC + c

        def zero(i, carry):
            deg_v[pl.ds(i * 16, 16)] = jnp.zeros((16,), jnp.float32)
            return carry
        lax.fori_loop(0, (N + 16) // 16, zero, 0)
        pltpu.sync_copy(dst_hbm.at[wid], idx_v)
        ones = jnp.full((16,), 1.0, jnp.float32)

        def body(j, carry):
            def inner(q, carry2):
                v = idx_v[j, pl.ds(q * 16, 16)]
                plsc.addupdate_scatter(deg_v, [v], ones)
                return carry2
            return lax.fori_loop(0, K // 16, inner, carry)

        lax.fori_loop(0, CPW, body, 0)
        pltpu.sync_copy(deg_v.at[pl.ds(0, N)], out_hbm.at[pl.ds(wid * N, N)])

    return deg(dst3)


def _acc_init(s, zeros_hbm, acc_sp):
    @pl.when(s < NS - 1)
    def _():
        pltpu.sync_copy(zeros_hbm.at[pl.ds(0, RF)],
                        acc_sp.at[pl.ds(s * RF, RF)])

    @pl.when(s == NS - 1)
    def _():
        pltpu.sync_copy(zeros_hbm.at[pl.ds(0, RL)],
                        acc_sp.at[pl.ds((NS - 1) * RF, RL)])


def _acc_out(c, s, acc_sp, out_hbm):
    @pl.when(s < NS - 1)
    def _():
        pltpu.sync_copy(acc_sp.at[pl.ds(s * RF, RF)],
                        out_hbm.at[c, pl.ds(s * RF, RF)])

    @pl.when(s == NS - 1)
    def _():
        pltpu.sync_copy(acc_sp.at[pl.ds((NS - 1) * RF, RL)],
                        out_hbm.at[c, pl.ds((NS - 1) * RF, RL)])


def _agg1_call(u2n, src_cat, dst16, zeros):
    @functools.partial(
        pl.kernel,
        out_type=jax.ShapeDtypeStruct((NC, N, 64), jnp.float32),
        mesh=_sc_mesh(),
        compiler_params=_untiled(),
        scratch_types=[
            pltpu.VMEM((CPT * K,), jnp.int32),
            pltpu.VMEM((CPT, K), jnp.int32),
            [pltpu.VMEM((K, 64), jnp.float32) for _ in range(NBUF)],
            pltpu.VMEM_SHARED((N + 8, 64), jnp.float32),
            [pltpu.SemaphoreType.DMA for _ in range(NBUF)],
        ],
    )
    def agg(u_hbm, src_hbm, dst_hbm, zeros_hbm, out_hbm,
            src_v, dst_v, rows, acc_sp, sems):
        c = lax.axis_index("c")
        s = lax.axis_index("s")
        _acc_init(s, zeros_hbm, acc_sp)
        pltpu.sync_copy(src_hbm.at[pl.ds(c * EP + s * CPT * K, CPT * K)],
                        src_v)
        pltpu.sync_copy(dst_hbm.at[s], dst_v)
        plsc.subcore_barrier()

        def gather(j, b):
            pltpu.async_copy(u_hbm.at[src_v.at[pl.ds(j * K, K)]],
                             rows[b], sems[b])

        def wait(b):
            pltpu.make_async_copy(u_hbm.at[src_v.at[pl.ds(0, K)]],
                                  rows[b], sems[b]).wait()

        def scatter(j, b):
            pltpu.sync_copy(rows[b], acc_sp.at[dst_v.at[j]], add=True)

        _pipeline(CPT, gather, wait, scatter)
        plsc.subcore_barrier()
        _acc_out(c, s, acc_sp, out_hbm)

    return agg(u2n, src_cat, dst16, zeros)


def _agg2_call(u, src1, dst3, zeros):
    @functools.partial(
        pl.kernel,
        out_type=jax.ShapeDtypeStruct((NC, N, 48), jnp.float32),
        mesh=_sc_mesh(),
        compiler_params=_untiled(),
        scratch_types=[
            pltpu.VMEM((CPW * K,), jnp.int32),
            pltpu.VMEM((CPW, K), jnp.int32),
            [pltpu.VMEM((K, 48), jnp.float32) for _ in range(NBUF)],
            pltpu.VMEM_SHARED((N + 8, 48), jnp.float32),
            [pltpu.SemaphoreType.DMA for _ in range(NBUF)],
        ],
    )
    def agg(u_hbm, src_hbm, dst_hbm, zeros_hbm, out_hbm,
            src_v, dst_v, rows, acc_sp, sems):
        c = lax.axis_index("c")
        s = lax.axis_index("s")
        wid = s * NC + c
        _acc_init(s, zeros_hbm, acc_sp)
        pltpu.sync_copy(src_hbm.at[pl.ds(wid * CPW * K, CPW * K)], src_v)
        pltpu.sync_copy(dst_hbm.at[wid], dst_v)
        plsc.subcore_barrier()

        def gather(j, b):
            pltpu.async_copy(u_hbm.at[src_v.at[pl.ds(j * K, K)]],
                             rows[b], sems[b])

        def wait(b):
            pltpu.make_async_copy(u_hbm.at[src_v.at[pl.ds(0, K)]],
                                  rows[b], sems[b]).wait()

        def scatter(j, b):
            pltpu.sync_copy(rows[b], acc_sp.at[dst_v.at[j]], add=True)

        _pipeline(CPW, gather, wait, scatter)
        plsc.subcore_barrier()
        _acc_out(c, s, acc_sp, out_hbm)

    return agg(u, src1, dst3, zeros)


def _tc_prep(degp, x):
    BN = 2000

    def body(degp_ref, x_ref, dis_ref, u1_ref):
        deg = jnp.sum(degp_ref[...], axis=1, keepdims=True) + 1.0
        dis = lax.rsqrt(deg)
        dis_ref[...] = dis
        u = x_ref[...] * dis
        u1_ref[0] = u[:, :64]
        u1_ref[1] = u[:, 64:]

    return pl.pallas_call(
        body,
        grid=(N // BN,),
        in_specs=[
            pl.BlockSpec((BN, NW), lambda i: (i, 0)),
            pl.BlockSpec((BN, 128), lambda i: (i, 0)),
        ],
        out_specs=[
            pl.BlockSpec((BN, 1), lambda i: (i, 0)),
            pl.BlockSpec((NC, BN, 64), lambda i: (0, i, 0)),
        ],
        out_shape=[
            jax.ShapeDtypeStruct((N, 1), jnp.float32),
            jax.ShapeDtypeStruct((NC, N, 64), jnp.float32),
        ],
    )(degp, x)


def _tc_mid(p, u1, dis, W1, b1r, W2p):
    BN = 1000

    def body(p_ref, u1_ref, dis_ref, W1_ref, b1_ref, W2_ref, u2_ref):
        z1 = jnp.concatenate(
            [p_ref[0] + u1_ref[0], p_ref[1] + u1_ref[1]], axis=1)
        z1 = z1 * dis_ref[...]
        h1 = jnp.dot(z1, W1_ref[...], preferred_element_type=jnp.float32)
        h1 = jnp.maximum(h1 + b1_ref[...], 0.0)
        g = jnp.dot(h1, W2_ref[...], preferred_element_type=jnp.float32)
        u2_ref[...] = g * dis_ref[...]

    return pl.pallas_call(
        body,
        grid=(N // BN,),
        in_specs=[
            pl.BlockSpec((NC, BN, 64), lambda i: (0, i, 0)),
            pl.BlockSpec((NC, BN, 64), lambda i: (0, i, 0)),
            pl.BlockSpec((BN, 1), lambda i: (i, 0)),
            pl.BlockSpec((128, 256), lambda i: (0, 0)),
            pl.BlockSpec((1, 256), lambda i: (0, 0)),
            pl.BlockSpec((256, 48), lambda i: (0, 0)),
        ],
        out_specs=pl.BlockSpec((BN, 48), lambda i: (i, 0)),
        out_shape=jax.ShapeDtypeStruct((N, 48), jnp.float32),
    )(p, u1, dis, W1, b1r, W2p)


def _tc_final(q, u2, dis, b2r):
    BN = 1000

    def body(q_ref, u2_ref, dis_ref, b2_ref, out_ref):
        z = (q_ref[0] + q_ref[1] + u2_ref[...]) * dis_ref[...]
        logits = z[:, :40] + b2_ref[...]
        m = jnp.max(logits, axis=1, keepdims=True)
        ex = jnp.exp(logits - m)
        lse = jnp.log(jnp.sum(ex, axis=1, keepdims=True)) + m
        out_ref[...] = logits - lse

    return pl.pallas_call(
        body,
        grid=(N // BN,),
        in_specs=[
            pl.BlockSpec((NC, BN, 48), lambda i: (0, i, 0)),
            pl.BlockSpec((BN, 48), lambda i: (i, 0)),
            pl.BlockSpec((BN, 1), lambda i: (i, 0)),
            pl.BlockSpec((1, 40), lambda i: (0, 0)),
        ],
        out_specs=pl.BlockSpec((BN, 40), lambda i: (i, 0)),
        out_shape=jax.ShapeDtypeStruct((N, 40), jnp.float32),
    )(q, u2, dis, b2r)


def kernel(x, edge, W1, b1, W2, b2):
    f32 = jnp.float32
    i32 = jnp.int32
    pad = EP - E
    srcp = jnp.concatenate([edge[0], jnp.zeros((pad,), i32)])
    dstp = jnp.concatenate([edge[1], jnp.full((pad,), N, i32)])
    src_cat = jnp.concatenate([srcp, srcp + N])
    dst16 = dstp.reshape(NS, CPT, K)
    dst32 = dstp.reshape(NW, CPW, K)

    degf = _deg_call(dst32)
    degp = degf.reshape(NW, N).T.reshape(N, NW)
    dis, u1 = _tc_prep(degp, x)
    p = _agg1_call(u1.reshape(NC * N, 64), src_cat, dst16,
                   jnp.zeros((RF, 64), f32))
    u2 = _tc_mid(p, u1, dis, W1, b1.reshape(1, -1),
                 jnp.pad(W2, ((0, 0), (0, 8))))
    q = _agg2_call(u2, srcp, dst32, jnp.zeros((RF, 48), f32))
    return _tc_final(q, u2, dis, b2.reshape(1, -1))
